# baseline (speedup 1.0000x reference)
import functools

import jax
import jax.numpy as jnp
from jax import lax
from jax.experimental import pallas as pl
from jax.experimental.pallas import tpu as pltpu
from jax.experimental.pallas import tpu_sc as plsc

B = 16384
N_DENSE = 13
N_SPARSE = 26
VOCAB = 100000
EMB = 32
X_DIM = N_DENSE + N_SPARSE * EMB
S_DIM = N_SPARSE * EMB

NSUB = 28
R2 = B * NSUB
NW = 32
RPW = R2 // NW
G = 128
NG = RPW // G
NBUF = 4
OROW = R2 // 4

BT = 512
NT = 7


def _sc_gather(table, idx2d):
    mesh = plsc.VectorSubcoreMesh(core_axis_name="c", subcore_axis_name="s")

    @functools.partial(
        pl.kernel,
        mesh=mesh,
        out_type=jax.ShapeDtypeStruct((R2, EMB), jnp.float32),
        scratch_types=[pltpu.VMEM((NG, G), jnp.int32),
                       pltpu.VMEM((NBUF, G, EMB), jnp.float32)]
                      + [pltpu.SemaphoreType.DMA] * NBUF,
        compiler_params=pltpu.CompilerParams(use_tc_tiling_on_sc=False),
    )
    def k(table_hbm, idx_hbm, out_hbm, idx_v, rows_v, *sems):
        wid = lax.axis_index("s") * 2 + lax.axis_index("c")
        cbase = wid * NG
        pltpu.sync_copy(idx_hbm.at[pl.ds(cbase, NG)], idx_v)
        for b in range(NBUF):
            pltpu.async_copy(table_hbm.at[idx_v.at[b]], rows_v.at[b], sems[b])

        def body(g, carry):
            for b in range(NBUF):
                j = g * NBUF + b
                pltpu.make_async_copy(out_hbm.at[pl.ds(0, G)], rows_v.at[b],
                                      sems[b]).wait()
                pltpu.sync_copy(rows_v.at[b],
                                out_hbm.at[pl.ds((cbase + j) * G, G)])
                nxt = j + NBUF

                @pl.when(nxt < NG)
                def _():
                    pltpu.async_copy(table_hbm.at[idx_v.at[nxt]],
                                     rows_v.at[b], sems[b])
            return carry

        lax.fori_loop(0, NG // NBUF, body, 0)

    return k(table, idx2d)


def _tc_body(dense_ref, sp_ref, w1_ref, wc_ref, w2_ref, w3_ref, wo_ref,
             b1_ref, b2_ref, b3_ref, cb_ref, bo_ref, out_ref):
    f32 = jnp.float32
    bf16 = jnp.bfloat16

    def bdot(a, b):
        return lax.dot_general(a.astype(bf16), b.astype(bf16),
                               (((1,), (0,)), ((), ())),
                               preferred_element_type=f32)

    v = sp_ref[...]
    pieces = [dense_ref[...]]
    for t in range(NT):
        pieces.append(v[:, t].reshape(BT, 128))
    x = jnp.concatenate(pieces, axis=1)[:, :X_DIM]

    cb = cb_ref[...]
    wc = wc_ref[...]

    xl = x
    for k in range(3):
        sk = bdot(xl, wc[:, k:k + 1])
        xl = x * sk + cb[:, k:k + 1] + xl

    h = jnp.maximum(bdot(x, w1_ref[...]) + b1_ref[...], 0.0)
    h = jnp.maximum(bdot(h, w2_ref[...]) + b2_ref[...], 0.0)
    h = jnp.maximum(bdot(h, w3_ref[...]) + b3_ref[...], 0.0)
    cat = jnp.concatenate([xl, h], axis=1)
    logit = bdot(cat, wo_ref[...]) + bo_ref[...]
    out_ref[...] = jax.nn.sigmoid(logit)


def kernel(inputs, embed_tables, cross_w, cross_b, W1, b1, W2, b2, W3, b3, Wo, bo):
    dense = inputs[:, :N_DENSE]
    idx = inputs[:, N_DENSE:].astype(jnp.int32)
    offs = (jnp.arange(N_SPARSE, dtype=jnp.int32) * VOCAB)[None, :]
    idxp = jnp.concatenate(
        [idx + offs, jnp.zeros((B, NSUB - N_SPARSE), jnp.int32)], axis=1)
    idx2d = (idxp.reshape(B // 8, 8, NT, 4)
             .transpose(0, 2, 1, 3).reshape(R2 // G, G))
    table = embed_tables.reshape(N_SPARSE * VOCAB, EMB)

    rows = _sc_gather(table, idx2d)
    sp4 = rows.reshape(B // 8, NT, 8, 128)

    wc = jnp.concatenate([cross_w[0], cross_w[1], cross_w[2]], axis=1)
    b1r, b2r, b3r = b1[None, :], b2[None, :], b3[None, :]
    cbr = cross_b.reshape(1, 3)
    bor = bo.reshape(1, 1)

    rep = lambda shape: pl.BlockSpec(shape, lambda i: (0,) * len(shape))
    out = pl.pallas_call(
        _tc_body,
        grid=(B // BT,),
        in_specs=[
            pl.BlockSpec((BT, N_DENSE), lambda i: (i, 0)),
            pl.BlockSpec((BT // 8, NT, 8, 128), lambda i: (i, 0, 0, 0)),
            rep((X_DIM, 256)), rep((X_DIM, 3)),
            rep((256, 128)), rep((128, 64)), rep((X_DIM + 64, 1)),
            rep((1, 256)), rep((1, 128)), rep((1, 64)),
            rep((1, 3)), rep((1, 1)),
        ],
        out_specs=pl.BlockSpec((BT, 1), lambda i: (i, 0)),
        out_shape=jax.ShapeDtypeStruct((B, 1), jnp.float32),
    )(dense, sp4, W1, wc, W2, W3, Wo, b1r, b2r, b3r, cbr, bor)
    return out


# speedup vs baseline: 1.2904x; 1.2904x over previous
import functools

import jax
import jax.numpy as jnp
from jax import lax
from jax.experimental import pallas as pl
from jax.experimental.pallas import tpu as pltpu
from jax.experimental.pallas import tpu_sc as plsc

B = 16384
N_DENSE = 13
N_SPARSE = 26
VOCAB = 100000
EMB = 32
X_DIM = N_DENSE + N_SPARSE * EMB
S_DIM = N_SPARSE * EMB

NSUB = 28
R2 = B * NSUB
NW = 32
RPW = R2 // NW
G = 128
NG = RPW // G
NBUF = 4
OROW = R2 // 4

BT = 512
NT = 7


def _sc_gather(table, idx2d):
    mesh = plsc.VectorSubcoreMesh(core_axis_name="c", subcore_axis_name="s")

    @functools.partial(
        pl.kernel,
        mesh=mesh,
        out_type=jax.ShapeDtypeStruct((R2, EMB), jnp.float32),
        scratch_types=[pltpu.VMEM((NG, G), jnp.int32),
                       pltpu.VMEM((NBUF, G, EMB), jnp.float32)]
                      + [pltpu.SemaphoreType.DMA] * NBUF,
        compiler_params=pltpu.CompilerParams(use_tc_tiling_on_sc=False),
    )
    def k(table_hbm, idx_hbm, out_hbm, idx_v, rows_v, *sems):
        wid = lax.axis_index("s") * 2 + lax.axis_index("c")
        cbase = wid * NG
        pltpu.sync_copy(idx_hbm.at[pl.ds(cbase, NG)], idx_v)
        for b in range(NBUF):
            pltpu.async_copy(table_hbm.at[idx_v.at[b]], rows_v.at[b], sems[b])

        def body(g, carry):
            for b in range(NBUF):
                j = g * NBUF + b
                pltpu.make_async_copy(out_hbm.at[pl.ds(0, G)], rows_v.at[b],
                                      sems[b]).wait()
                pltpu.sync_copy(rows_v.at[b],
                                out_hbm.at[pl.ds((cbase + j) * G, G)])
                nxt = j + NBUF

                @pl.when(nxt < NG)
                def _():
                    pltpu.async_copy(table_hbm.at[idx_v.at[nxt]],
                                     rows_v.at[b], sems[b])
            return carry

        lax.fori_loop(0, NG // NBUF, body, 0)

    return k(table, idx2d)


def _tc_body(dense_ref, sp_ref, w1_ref, wc_ref, w2_ref, w3_ref, wo_ref,
             b1_ref, b2_ref, b3_ref, cb_ref, bo_ref, out_ref):
    f32 = jnp.float32
    bf16 = jnp.bfloat16

    def bdot(a, b):
        return lax.dot_general(a.astype(bf16), b.astype(bf16),
                               (((1,), (0,)), ((), ())),
                               preferred_element_type=f32)

    v = sp_ref[...]
    pieces = [dense_ref[...]]
    for t in range(NT):
        pieces.append(v[:, t].reshape(BT, 128))
    x = jnp.concatenate(pieces, axis=1)[:, :X_DIM]

    cb = cb_ref[...]
    wc = wc_ref[...]

    xl = x
    for k in range(3):
        sk = bdot(xl, wc[:, k:k + 1])
        xl = x * sk + cb[:, k:k + 1] + xl

    h = jnp.maximum(bdot(x, w1_ref[...]) + b1_ref[...], 0.0)
    h = jnp.maximum(bdot(h, w2_ref[...]) + b2_ref[...], 0.0)
    h = jnp.maximum(bdot(h, w3_ref[...]) + b3_ref[...], 0.0)
    cat = jnp.concatenate([xl, h], axis=1)
    logit = bdot(cat, wo_ref[...]) + bo_ref[...]
    out_ref[...] = jax.nn.sigmoid(logit)


def kernel(inputs, embed_tables, cross_w, cross_b, W1, b1, W2, b2, W3, b3, Wo, bo):
    dense = inputs[:, :N_DENSE]
    idx = inputs[:, N_DENSE:].astype(jnp.int32)
    offs = (jnp.arange(N_SPARSE, dtype=jnp.int32) * VOCAB)[None, :]
    idxp = jnp.concatenate(
        [idx + offs, idx[:, :NSUB - N_SPARSE] + offs[:, :NSUB - N_SPARSE]],
        axis=1)
    idx2d = (idxp.reshape(B // 8, 8, NT, 4)
             .transpose(0, 2, 1, 3).reshape(R2 // G, G))
    table = embed_tables.reshape(N_SPARSE * VOCAB, EMB)

    rows = _sc_gather(table, idx2d)
    sp4 = rows.reshape(B // 8, NT, 8, 128)

    wc = jnp.concatenate([cross_w[0], cross_w[1], cross_w[2]], axis=1)
    b1r, b2r, b3r = b1[None, :], b2[None, :], b3[None, :]
    cbr = cross_b.reshape(1, 3)
    bor = bo.reshape(1, 1)

    rep = lambda shape: pl.BlockSpec(shape, lambda i: (0,) * len(shape))
    out = pl.pallas_call(
        _tc_body,
        grid=(B // BT,),
        in_specs=[
            pl.BlockSpec((BT, N_DENSE), lambda i: (i, 0)),
            pl.BlockSpec((BT // 8, NT, 8, 128), lambda i: (i, 0, 0, 0)),
            rep((X_DIM, 256)), rep((X_DIM, 3)),
            rep((256, 128)), rep((128, 64)), rep((X_DIM + 64, 1)),
            rep((1, 256)), rep((1, 128)), rep((1, 64)),
            rep((1, 3)), rep((1, 1)),
        ],
        out_specs=pl.BlockSpec((BT, 1), lambda i: (i, 0)),
        out_shape=jax.ShapeDtypeStruct((B, 1), jnp.float32),
    )(dense, sp4, W1, wc, W2, W3, Wo, b1r, b2r, b3r, cbr, bor)
    return out
